# baseline (device time: 106607 ns/iter reference)
import jax
import jax.numpy as jnp
from jax import lax
from jax.experimental import pallas as pl
from jax.experimental.pallas import tpu as pltpu

N_DEV = 8
M = 512
K = 4096
N = 8192
NB = N // N_DEV
CW = 512
SPC = NB // CW
NSUB = N // CW

E4M3_MAX = 448.0

QBF = 5


def kernel(x, w_mat):
    assert x.shape == (M, K), x.shape
    assert w_mat.shape == (K, N), w_mat.shape
    x = x.astype(jnp.bfloat16)

    def body(x_ref, w_ref, out_ref, qsend_ref, wbuf_ref, qrecv_ref,
             qse4_ref, qre4_ref, amax_tx_ref, amax_ref,
             wsems, send_sems, recv_sems, amax_send_sems, amax_recv_sems):
        me = lax.axis_index("i")

        bsem = pltpu.get_barrier_semaphore()
        for t in range(N_DEV):
            @pl.when(me != t)
            def _():
                pl.semaphore_signal(
                    bsem, inc=1, device_id=(t,),
                    device_id_type=pl.DeviceIdType.MESH,
                )
        pl.semaphore_wait(bsem, N_DEV - 1)

        def w_col(k):
            c = k // SPC
            tc = (me + 1 + c) % N_DEV
            return tc * NB + (k % SPC) * CW

        def start_w(k):
            pltpu.make_async_copy(
                w_ref.at[:, pl.ds(w_col(k), CW)],
                wbuf_ref.at[k % 2],
                wsems.at[k % 2],
            ).start()

        def wait_w(k):
            pltpu.make_async_copy(
                w_ref.at[:, pl.ds(w_col(k), CW)],
                wbuf_ref.at[k % 2],
                wsems.at[k % 2],
            ).wait()

        start_w(0)
        am = jnp.float32(0.0)
        for k in range(NSUB):
            c, h = k // SPC, k % SPC
            if k + 1 < NSUB:
                start_w(k + 1)
            wait_w(k)
            yc = jnp.dot(
                x_ref[...], wbuf_ref[k % 2].astype(jnp.bfloat16),
                preferred_element_type=jnp.float32,
            )
            qsend_ref[c, :, pl.ds(h * CW, CW)] = yc.astype(jnp.bfloat16)
            am = jnp.maximum(am, jnp.max(yc))

            if h == SPC - 1 and c < QBF:
                tc = (me + 1 + c) % N_DEV
                pltpu.make_async_remote_copy(
                    src_ref=qsend_ref.at[pl.ds(c, 1)],
                    dst_ref=qrecv_ref.at[pl.ds(c, 1)],
                    send_sem=send_sems.at[c],
                    recv_sem=recv_sems.at[c],
                    device_id=(tc,),
                    device_id_type=pl.DeviceIdType.MESH,
                ).start()

        amax_tx_ref[...] = jnp.full((1, 128), am, jnp.float32)
        amax_ref[0, :] = amax_tx_ref[0, :]
        for c in range(1, N_DEV):
            t = (me + c) % N_DEV
            pltpu.make_async_remote_copy(
                src_ref=amax_tx_ref,
                dst_ref=amax_ref.at[pl.ds(c, 1)],
                send_sem=amax_send_sems.at[c],
                recv_sem=amax_recv_sems.at[c],
                device_id=(t,),
                device_id_type=pl.DeviceIdType.MESH,
            ).start()
        for c in range(1, N_DEV):
            pltpu.make_async_remote_copy(
                src_ref=amax_tx_ref,
                dst_ref=amax_ref.at[pl.ds(c, 1)],
                send_sem=amax_send_sems.at[c],
                recv_sem=amax_recv_sems.at[c],
                device_id=(0,),
                device_id_type=pl.DeviceIdType.MESH,
            ).wait_recv()
        g_amax = jnp.max(amax_ref[...])
        scale = g_amax / E4M3_MAX
        inv = jnp.where(g_amax > 0.0, E4M3_MAX / g_amax, 0.0)

        def quant_dequant(v):
            q = jnp.minimum(jnp.maximum(v, 0.0) * inv, E4M3_MAX)
            return q.astype(jnp.float8_e4m3fn).astype(jnp.float32) * scale

        for c in range(QBF, N_DEV - 1):
            tc = (me + 1 + c) % N_DEV
            v = qsend_ref[c].astype(jnp.float32)
            qse4_ref[c - QBF] = jnp.minimum(
                jnp.maximum(v, 0.0) * inv, E4M3_MAX
            ).astype(jnp.float8_e4m3fn)
            pltpu.make_async_remote_copy(
                src_ref=qse4_ref.at[pl.ds(c - QBF, 1)],
                dst_ref=qre4_ref.at[pl.ds(c - QBF, 1)],
                send_sem=send_sems.at[c],
                recv_sem=recv_sems.at[c],
                device_id=(tc,),
                device_id_type=pl.DeviceIdType.MESH,
            ).start()

        own = qsend_ref[N_DEV - 1].astype(jnp.float32)
        out_ref[pl.ds(me * M, M), :] = quant_dequant(own)

        for c in range(N_DEV - 1):
            s = (me - 1 - c) % N_DEV
            if c < QBF:
                pltpu.make_async_remote_copy(
                    src_ref=qsend_ref.at[pl.ds(c, 1)],
                    dst_ref=qrecv_ref.at[pl.ds(c, 1)],
                    send_sem=send_sems.at[c],
                    recv_sem=recv_sems.at[c],
                    device_id=(0,),
                    device_id_type=pl.DeviceIdType.MESH,
                ).wait_recv()
                out_ref[pl.ds(s * M, M), :] = quant_dequant(
                    qrecv_ref[c].astype(jnp.float32))
            else:
                pltpu.make_async_remote_copy(
                    src_ref=qse4_ref.at[pl.ds(c - QBF, 1)],
                    dst_ref=qre4_ref.at[pl.ds(c - QBF, 1)],
                    send_sem=send_sems.at[c],
                    recv_sem=recv_sems.at[c],
                    device_id=(0,),
                    device_id_type=pl.DeviceIdType.MESH,
                ).wait_recv()
                out_ref[pl.ds(s * M, M), :] = (
                    qre4_ref[c - QBF].astype(jnp.float32) * scale)

        for c in range(N_DEV - 1):
            if c < QBF:
                pltpu.make_async_remote_copy(
                    src_ref=qsend_ref.at[pl.ds(c, 1)],
                    dst_ref=qrecv_ref.at[pl.ds(c, 1)],
                    send_sem=send_sems.at[c],
                    recv_sem=recv_sems.at[c],
                    device_id=(0,),
                    device_id_type=pl.DeviceIdType.MESH,
                ).wait_send()
            else:
                pltpu.make_async_remote_copy(
                    src_ref=qse4_ref.at[pl.ds(c - QBF, 1)],
                    dst_ref=qre4_ref.at[pl.ds(c - QBF, 1)],
                    send_sem=send_sems.at[c],
                    recv_sem=recv_sems.at[c],
                    device_id=(0,),
                    device_id_type=pl.DeviceIdType.MESH,
                ).wait_send()
        for c in range(1, N_DEV):
            pltpu.make_async_remote_copy(
                src_ref=amax_tx_ref,
                dst_ref=amax_ref.at[pl.ds(c, 1)],
                send_sem=amax_send_sems.at[c],
                recv_sem=amax_recv_sems.at[c],
                device_id=(0,),
                device_id_type=pl.DeviceIdType.MESH,
            ).wait_send()

    return pl.pallas_call(
        body,
        in_specs=[
            pl.BlockSpec(memory_space=pltpu.VMEM),
            pl.BlockSpec(memory_space=pl.ANY),
        ],
        out_specs=pl.BlockSpec(memory_space=pltpu.VMEM),
        out_shape=jax.ShapeDtypeStruct((N_DEV * M, NB), jnp.float32),
        scratch_shapes=[
            pltpu.VMEM((N_DEV, M, NB), jnp.bfloat16),
            pltpu.VMEM((2, K, CW), jnp.float32),
            pltpu.VMEM((N_DEV, M, NB), jnp.bfloat16),
            pltpu.VMEM((2, M, NB), jnp.float8_e4m3fn),
            pltpu.VMEM((2, M, NB), jnp.float8_e4m3fn),
            pltpu.VMEM((1, 128), jnp.float32),
            pltpu.VMEM((N_DEV, 128), jnp.float32),
            pltpu.SemaphoreType.DMA((2,)),
            pltpu.SemaphoreType.DMA((N_DEV,)),
            pltpu.SemaphoreType.DMA((N_DEV,)),
            pltpu.SemaphoreType.DMA((N_DEV,)),
            pltpu.SemaphoreType.DMA((N_DEV,)),
        ],
        compiler_params=pltpu.CompilerParams(
            collective_id=0,
            vmem_limit_bytes=64 * 1024 * 1024,
        ),
    )(x, w_mat)
